# baseline (speedup 1.0000x reference)
import jax
import jax.numpy as jnp
from jax import lax
from jax.experimental import pallas as pl
from jax.experimental.pallas import tpu as pltpu
from jax.experimental.pallas import tpu_sc as plsc

N_VOX = 50000
PTS = 35
FEAT = 8
LANES = 16

NUM_WORKERS = 32
TILE_VOX = 128
FULL_TILES = N_VOX // TILE_VOX
TAIL_VOX = N_VOX - FULL_TILES * TILE_VOX
K_ITERS = -(-FULL_TILES // NUM_WORKERS)
NBUF = 4
SLOT_ROWS = 40
LUT_SIZE = 528


def _sc_body(
    feat_hbm,
    out_hbm,
    big_v,
    labt_v,
    outc_v,
    lut_v,
    in_sem0,
    in_sem1,
    in_sem2,
    in_sem3,
    out_sem0,
    out_sem1,
):
    cid = lax.axis_index("c")
    sid = lax.axis_index("s")
    wid = sid * 2 + cid

    in_sems = (in_sem0, in_sem1, in_sem2, in_sem3)
    out_sems = (out_sem0, out_sem1)

    zeros = jnp.zeros((LANES,), jnp.int32)
    ones = jnp.ones((LANES,), jnp.int32)
    sevens = jnp.full((LANES,), 7, jnp.int32)
    iota = lax.iota(jnp.int32, LANES)

    for j in range(LUT_SIZE // LANES):
        lut_v[pl.ds(j * LANES, LANES)] = zeros
    lut_v[pl.ds(0, LANES)] = jnp.where(iota == 0, 1, 0).astype(jnp.int32)
    lut_v[pl.ds(496, LANES)] = jnp.where(iota == 12, 16, 0).astype(jnp.int32)
    lut_v[pl.ds(512, LANES)] = jnp.where(
        iota == 0,
        1 << 8,
        jnp.where(
            iota == 2,
            1 << 12,
            jnp.where((iota >= 4) & (iota <= 7), ones << (iota << 2), 0),
        ),
    ).astype(jnp.int32)

    def histo_group(load_row, store_row, g):
        cnt = [zeros] * FEAT
        for blk_start in range(0, PTS, 15):
            blk_end = min(blk_start + 15, PTS)
            acc = zeros
            for p in range(blk_start, blk_end):
                bits = lax.bitcast_convert_type(load_row(p, g), jnp.int32)
                acc = acc + plsc.load_gather(lut_v, [bits >> 21])
            for c in range(1, FEAT):
                cnt[c] = cnt[c] + ((acc >> (4 * c)) & 15)
        best = sevens
        for c in range(1, FEAT):
            best = jnp.maximum(best, (cnt[c] << 3) + (7 - c))
        store_row(g, 7 - (best & 7))

    def tile_of(k):
        return k * NUM_WORKERS + wid

    def for_static(b, n, fn):
        for bs in range(n):
            @pl.when(b == bs)
            def _(bs=bs):
                fn(bs)

    def start_in(k, bs):
        pltpu.async_copy(
            feat_hbm.at[:, 7, pl.ds(tile_of(k) * TILE_VOX, TILE_VOX)],
            big_v.at[pl.ds(bs * SLOT_ROWS, PTS), :],
            in_sems[bs],
        )

    def wait_in(k, bs):
        pltpu.make_async_copy(
            feat_hbm.at[:, 7, pl.ds(tile_of(k) * TILE_VOX, TILE_VOX)],
            big_v.at[pl.ds(bs * SLOT_ROWS, PTS), :],
            in_sems[bs],
        ).wait()

    def start_out(k, bs):
        pltpu.async_copy(
            outc_v.at[bs * 8, :],
            out_hbm.at[pl.ds(tile_of(k) * TILE_VOX, TILE_VOX)],
            out_sems[bs],
        )

    def wait_out(k, bs):
        pltpu.make_async_copy(
            outc_v.at[bs * 8, :],
            out_hbm.at[pl.ds(tile_of(k) * TILE_VOX, TILE_VOX)],
            out_sems[bs],
        ).wait()

    for kp in range(NBUF - 1):
        start_in(kp, kp)

    def round_body(k, carry):
        s = k & (NBUF - 1)
        bo = k & 1

        @pl.when(tile_of(k + NBUF - 1) < FULL_TILES)
        def _():
            for_static(
                (k + NBUF - 1) & (NBUF - 1),
                NBUF,
                lambda bs: start_in(k + NBUF - 1, bs),
            )

        @pl.when(tile_of(k) < FULL_TILES)
        def _():
            for_static(s, NBUF, lambda bs: wait_in(k, bs))

            @pl.when(k >= 2)
            def _():
                for_static(bo, 2, lambda bs: wait_out(k - 2, bs))

            srow = s * SLOT_ROWS
            orow = bo * 8

            def load_row(p, g):
                return big_v[srow + p, pl.ds(g * LANES, LANES)]

            def store_row(g, x):
                outc_v[orow, pl.ds(g * LANES, LANES)] = x

            def group_body(g, c2):
                histo_group(load_row, store_row, g)
                return c2

            lax.fori_loop(0, TILE_VOX // LANES, group_body, 0)
            for_static(bo, 2, lambda bs: start_out(k, bs))

        return carry

    lax.fori_loop(0, K_ITERS, round_body, 0)

    r_final = jnp.where(tile_of(K_ITERS - 1) < FULL_TILES, K_ITERS, K_ITERS - 1)
    for b in range(2):
        kb = r_final - 2 + ((r_final + b) & 1)
        wait_out(kb, b)

    @pl.when(wid == NUM_WORKERS - 1)
    def _():
        vox_base = FULL_TILES * TILE_VOX
        pltpu.sync_copy(feat_hbm.at[:, 7, pl.ds(vox_base, TAIL_VOX)], labt_v)

        def tload_row(p, g):
            return labt_v[p, pl.ds(g * LANES, LANES)]

        def tstore_row(g, x):
            outc_v[0, pl.ds(g * LANES, LANES)] = x

        def tail_group(g, c2):
            histo_group(tload_row, tstore_row, g)
            return c2

        lax.fori_loop(0, TAIL_VOX // LANES, tail_group, 0)
        pltpu.sync_copy(
            outc_v.at[0, pl.ds(0, TAIL_VOX)],
            out_hbm.at[pl.ds(vox_base, TAIL_VOX)],
        )


_sc_call = pl.kernel(
    _sc_body,
    out_type=jax.ShapeDtypeStruct((N_VOX,), jnp.int32),
    mesh=plsc.VectorSubcoreMesh(core_axis_name="c", subcore_axis_name="s"),
    scratch_types=[
        pltpu.VMEM((NBUF * SLOT_ROWS, TILE_VOX), jnp.float32),
        pltpu.VMEM((PTS, TAIL_VOX), jnp.float32),
        pltpu.VMEM((16, TILE_VOX), jnp.int32),
        pltpu.VMEM((LUT_SIZE,), jnp.int32),
        pltpu.SemaphoreType.DMA,
        pltpu.SemaphoreType.DMA,
        pltpu.SemaphoreType.DMA,
        pltpu.SemaphoreType.DMA,
        pltpu.SemaphoreType.DMA,
        pltpu.SemaphoreType.DMA,
    ],
    compiler_params=pltpu.CompilerParams(
        needs_layout_passes=False, use_tc_tiling_on_sc=True
    ),
)


@jax.jit
def kernel(features):
    t = jnp.transpose(features, (1, 2, 0))
    out = _sc_call(t)
    return out.reshape(-1, 1).astype(jnp.int64)


# speedup vs baseline: 1.1544x; 1.1544x over previous
import jax
import jax.numpy as jnp
from jax import lax
from jax.experimental import pallas as pl
from jax.experimental.pallas import tpu as pltpu
from jax.experimental.pallas import tpu_sc as plsc

N_VOX = 50000
PTS = 35
FEAT = 8
LANES = 16

NUM_WORKERS = 32
TILE_VOX = 128
FULL_TILES = N_VOX // TILE_VOX
TAIL_VOX = N_VOX - FULL_TILES * TILE_VOX
K_ITERS = -(-FULL_TILES // NUM_WORKERS)
NBUF = 4
SLOT_ROWS = 40


def _sc_body(
    feat_hbm,
    out_hbm,
    big_v,
    labt_v,
    outc_v,
    in_sem0,
    in_sem1,
    in_sem2,
    in_sem3,
    out_sem0,
    out_sem1,
):
    cid = lax.axis_index("c")
    sid = lax.axis_index("s")
    wid = sid * 2 + cid

    in_sems = (in_sem0, in_sem1, in_sem2, in_sem3)
    out_sems = (out_sem0, out_sem1)

    zeros = jnp.zeros((LANES,), jnp.int32)
    ones = jnp.ones((LANES,), jnp.int32)
    sevens = jnp.full((LANES,), 7, jnp.int32)

    nibbles = jnp.full((LANES,), 0x0F0F0F0F, jnp.int32)

    def histo_group(load_row, store_row, g):
        accs = []
        for blk_start in range(0, PTS, 15):
            blk_end = min(blk_start + 15, PTS)
            acc = zeros
            for p in range(blk_start, blk_end):
                lbl = load_row(p, g).astype(jnp.int32)
                acc = acc + (ones << (lbl << 2))
            accs.append(acc)
        ev = (accs[0] & nibbles) + (accs[1] & nibbles) + (accs[2] & nibbles)
        od = (
            ((accs[0] >> 4) & nibbles)
            + ((accs[1] >> 4) & nibbles)
            + ((accs[2] >> 4) & nibbles)
        )
        best = sevens
        for c in range(1, FEAT):
            src = od if c & 1 else ev
            cnt = (src >> (8 * (c >> 1))) & 255
            best = jnp.maximum(best, (cnt << 3) + (7 - c))
        store_row(g, 7 - (best & 7))

    def tile_of(k):
        return k * NUM_WORKERS + wid

    def for_static(b, n, fn):
        for bs in range(n):
            @pl.when(b == bs)
            def _(bs=bs):
                fn(bs)

    def start_in(k, bs):
        pltpu.async_copy(
            feat_hbm.at[:, 7, pl.ds(tile_of(k) * TILE_VOX, TILE_VOX)],
            big_v.at[pl.ds(bs * SLOT_ROWS, PTS), :],
            in_sems[bs],
        )

    def wait_in(k, bs):
        pltpu.make_async_copy(
            feat_hbm.at[:, 7, pl.ds(tile_of(k) * TILE_VOX, TILE_VOX)],
            big_v.at[pl.ds(bs * SLOT_ROWS, PTS), :],
            in_sems[bs],
        ).wait()

    def start_out(k, bs):
        pltpu.async_copy(
            outc_v.at[bs * 8, :],
            out_hbm.at[pl.ds(tile_of(k) * TILE_VOX, TILE_VOX)],
            out_sems[bs],
        )

    def wait_out(k, bs):
        pltpu.make_async_copy(
            outc_v.at[bs * 8, :],
            out_hbm.at[pl.ds(tile_of(k) * TILE_VOX, TILE_VOX)],
            out_sems[bs],
        ).wait()

    for kp in range(NBUF - 1):
        start_in(kp, kp)

    def round_body(k, carry):
        s = k & (NBUF - 1)
        bo = k & 1

        @pl.when(tile_of(k + NBUF - 1) < FULL_TILES)
        def _():
            for_static(
                (k + NBUF - 1) & (NBUF - 1),
                NBUF,
                lambda bs: start_in(k + NBUF - 1, bs),
            )

        @pl.when(tile_of(k) < FULL_TILES)
        def _():
            for_static(s, NBUF, lambda bs: wait_in(k, bs))

            @pl.when(k >= 2)
            def _():
                for_static(bo, 2, lambda bs: wait_out(k - 2, bs))

            srow = s * SLOT_ROWS
            orow = bo * 8

            def load_row(p, g):
                return big_v[srow + p, pl.ds(g * LANES, LANES)]

            def store_row(g, x):
                outc_v[orow, pl.ds(g * LANES, LANES)] = x

            def group_body(g, c2):
                histo_group(load_row, store_row, g)
                return c2

            lax.fori_loop(0, TILE_VOX // LANES, group_body, 0)
            for_static(bo, 2, lambda bs: start_out(k, bs))

        return carry

    lax.fori_loop(0, K_ITERS, round_body, 0)

    r_final = jnp.where(tile_of(K_ITERS - 1) < FULL_TILES, K_ITERS, K_ITERS - 1)
    for b in range(2):
        kb = r_final - 2 + ((r_final + b) & 1)
        wait_out(kb, b)

    @pl.when(wid == NUM_WORKERS - 1)
    def _():
        vox_base = FULL_TILES * TILE_VOX
        pltpu.sync_copy(feat_hbm.at[:, 7, pl.ds(vox_base, TAIL_VOX)], labt_v)

        def tload_row(p, g):
            return labt_v[p, pl.ds(g * LANES, LANES)]

        def tstore_row(g, x):
            outc_v[0, pl.ds(g * LANES, LANES)] = x

        def tail_group(g, c2):
            histo_group(tload_row, tstore_row, g)
            return c2

        lax.fori_loop(0, TAIL_VOX // LANES, tail_group, 0)
        pltpu.sync_copy(
            outc_v.at[0, pl.ds(0, TAIL_VOX)],
            out_hbm.at[pl.ds(vox_base, TAIL_VOX)],
        )


_sc_call = pl.kernel(
    _sc_body,
    out_type=jax.ShapeDtypeStruct((N_VOX,), jnp.int32),
    mesh=plsc.VectorSubcoreMesh(core_axis_name="c", subcore_axis_name="s"),
    scratch_types=[
        pltpu.VMEM((NBUF * SLOT_ROWS, TILE_VOX), jnp.float32),
        pltpu.VMEM((PTS, TAIL_VOX), jnp.float32),
        pltpu.VMEM((16, TILE_VOX), jnp.int32),
        pltpu.SemaphoreType.DMA,
        pltpu.SemaphoreType.DMA,
        pltpu.SemaphoreType.DMA,
        pltpu.SemaphoreType.DMA,
        pltpu.SemaphoreType.DMA,
        pltpu.SemaphoreType.DMA,
    ],
    compiler_params=pltpu.CompilerParams(
        needs_layout_passes=False, use_tc_tiling_on_sc=True
    ),
)


@jax.jit
def kernel(features):
    t = jnp.transpose(features, (1, 2, 0))
    out = _sc_call(t)
    return out.reshape(-1, 1).astype(jnp.int64)
